# baseline (speedup 1.0000x reference)
import functools

import jax
import jax.numpy as jnp
from jax.experimental import pallas as pl
from jax.experimental.pallas import tpu as pltpu

_ANTIGEN_IDX = 2
_TEMPERATURE = 0.15


def _fused_body(chain_ref, s_ref, out_ref, norm_ref, cnt_ref):
    b = pl.program_id(0)
    bsz = pl.num_programs(0)
    s = s_ref[0]
    n_tok = s.shape[0]
    chain_row = chain_ref[b, 0, :]
    m = (chain_row == _ANTIGEN_IDX).astype(jnp.float32).reshape(n_tok, 1)
    tot = jnp.sum(s, axis=0)
    ag_s = jnp.sum(s * m, axis=0)
    ab_s = tot - ag_s

    ag_cnt = jnp.sum(m)
    ab_cnt = n_tok - ag_cnt
    cnt_ref[b] = ag_cnt

    ab_emb = ab_s / jnp.maximum(ab_cnt, 1.0)
    ag_emb = ag_s / jnp.maximum(ag_cnt, 1.0)
    ab_n = ab_emb / jnp.maximum(
        jnp.sqrt(jnp.sum(ab_emb * ab_emb)), 1e-12)
    ag_n = ag_emb / jnp.maximum(
        jnp.sqrt(jnp.sum(ag_emb * ag_emb)), 1e-12)
    norm_ref[b] = jnp.stack([ab_n, ag_n], axis=0)

    @pl.when(b == bsz - 1)
    def _loss():
        ab_all = norm_ref[:, 0, :]
        ag_all = norm_ref[:, 1, :]
        sim = jax.lax.dot_general(
            ab_all, ag_all, (((1,), (1,)), ((), ())),
            preferred_element_type=jnp.float32,
            precision=jax.lax.Precision.HIGHEST,
        ) / _TEMPERATURE

        ag_cnts = jnp.stack([cnt_ref[i] for i in range(bsz)])
        valid = ag_cnts > 0.0
        neg_inf = jnp.asarray(-jnp.inf, dtype=sim.dtype)
        sim_m = jnp.where(valid[None, :], sim, neg_inf)
        mx = jnp.max(sim_m, axis=1, keepdims=True)
        mx_safe = jnp.where(jnp.isfinite(mx), mx, 0.0)
        lse = jnp.log(
            jnp.sum(jnp.exp(sim_m - mx_safe), axis=1, keepdims=True)) + mx

        eye = (jax.lax.broadcasted_iota(jnp.int32, sim.shape, 0)
               == jax.lax.broadcasted_iota(jnp.int32, sim.shape, 1))
        logp = sim - lse
        diag = jnp.sum(jnp.where(eye, logp, 0.0), axis=1)

        n_valid = jnp.sum(valid.astype(jnp.float32))
        loss = -jnp.sum(jnp.where(valid, diag, 0.0)) / n_valid
        out_ref[...] = loss[None, None]


@functools.partial(jax.jit, static_argnames=("interpret",))
def kernel(s_i, chain_type, interpret=False):
    bsz, n_tok, dim = s_i.shape
    chain3 = chain_type.reshape(bsz, 1, n_tok)

    loss = pl.pallas_call(
        _fused_body,
        grid=(bsz,),
        in_specs=[
            pl.BlockSpec((bsz, 1, n_tok), lambda b: (0, 0, 0)),
            pl.BlockSpec((1, n_tok, dim), lambda b: (b, 0, 0)),
        ],
        out_specs=pl.BlockSpec((1, 1), lambda b: (0, 0)),
        out_shape=jax.ShapeDtypeStruct((1, 1), jnp.float32),
        scratch_shapes=[
            pltpu.VMEM((bsz, 2, dim), jnp.float32),
            pltpu.SMEM((bsz,), jnp.float32),
        ],
        interpret=interpret,
    )(chain3, s_i)

    return loss[0, 0]


# speedup vs baseline: 1.0263x; 1.0263x over previous
import functools

import jax
import jax.numpy as jnp
from jax.experimental import pallas as pl
from jax.experimental.pallas import tpu as pltpu

_ANTIGEN_IDX = 2
_TEMPERATURE = 0.15


def _fused_body(chain_ref, s_ref, out_ref, acc_ref):
    b = pl.program_id(0)
    bsz = pl.num_programs(0)
    s = s_ref[0]
    n_tok = s.shape[0]
    chain_row = chain_ref[b, 0, :]
    m = (chain_row == _ANTIGEN_IDX).astype(jnp.float32).reshape(n_tok, 1)
    tot = jnp.sum(s, axis=0)
    ag = jnp.sum(s * m, axis=0)
    acc_ref[b] = jnp.stack([tot, ag], axis=0)

    @pl.when(b == bsz - 1)
    def _loss():
        pooled = acc_ref[...]
        mask_all = (chain_ref[:, 0, :] == _ANTIGEN_IDX).astype(jnp.float32)
        ag_cnt = jnp.sum(mask_all, axis=1)
        ab_cnt = n_tok - ag_cnt

        tot_s = pooled[:, 0, :]
        ag_s = pooled[:, 1, :]
        ab_s = tot_s - ag_s

        ab_emb = ab_s / jnp.maximum(ab_cnt, 1.0)[:, None]
        ag_emb = ag_s / jnp.maximum(ag_cnt, 1.0)[:, None]

        ab_n = ab_emb / jnp.maximum(
            jnp.sqrt(jnp.sum(ab_emb * ab_emb, axis=1, keepdims=True)), 1e-12)
        ag_n = ag_emb / jnp.maximum(
            jnp.sqrt(jnp.sum(ag_emb * ag_emb, axis=1, keepdims=True)), 1e-12)

        sim = jax.lax.dot_general(
            ab_n, ag_n, (((1,), (1,)), ((), ())),
            preferred_element_type=jnp.float32,
            precision=jax.lax.Precision.HIGHEST,
        ) / _TEMPERATURE

        valid = ag_cnt > 0.0
        neg_inf = jnp.asarray(-jnp.inf, dtype=sim.dtype)
        sim_m = jnp.where(valid[None, :], sim, neg_inf)
        mx = jnp.max(sim_m, axis=1, keepdims=True)
        mx_safe = jnp.where(jnp.isfinite(mx), mx, 0.0)
        lse = jnp.log(
            jnp.sum(jnp.exp(sim_m - mx_safe), axis=1, keepdims=True)) + mx

        eye = (jax.lax.broadcasted_iota(jnp.int32, sim.shape, 0)
               == jax.lax.broadcasted_iota(jnp.int32, sim.shape, 1))
        logp = sim - lse
        diag = jnp.sum(jnp.where(eye, logp, 0.0), axis=1)

        n_valid = jnp.sum(valid.astype(jnp.float32))
        loss = -jnp.sum(jnp.where(valid, diag, 0.0)) / n_valid
        out_ref[...] = loss[None, None]


@functools.partial(jax.jit, static_argnames=("interpret",))
def kernel(s_i, chain_type, interpret=False):
    bsz, n_tok, dim = s_i.shape
    chain3 = chain_type.reshape(bsz, 1, n_tok)

    loss = pl.pallas_call(
        _fused_body,
        grid=(bsz,),
        in_specs=[
            pl.BlockSpec((bsz, 1, n_tok), lambda b: (0, 0, 0)),
            pl.BlockSpec((1, n_tok, dim), lambda b: (b, 0, 0)),
        ],
        out_specs=pl.BlockSpec((1, 1), lambda b: (0, 0)),
        out_shape=jax.ShapeDtypeStruct((1, 1), jnp.float32),
        scratch_shapes=[pltpu.VMEM((bsz, 2, dim), jnp.float32)],
        interpret=interpret,
    )(chain3, s_i)

    return loss[0, 0]


# speedup vs baseline: 1.2165x; 1.1854x over previous
import functools

import jax
import jax.numpy as jnp
from jax.experimental import pallas as pl
from jax.experimental.pallas import tpu as pltpu

_ANTIGEN_IDX = 2
_TEMPERATURE = 0.15
_N_PIPE = 2


def _fused_body(chain_ref, *refs):
    s_refs = refs[:_N_PIPE]
    out_ref = refs[_N_PIPE]
    acc_ref = refs[_N_PIPE + 1]
    b = pl.program_id(0)
    n_steps = pl.num_programs(0)
    bsz = chain_ref.shape[0]
    n_tok = chain_ref.shape[1]

    for p, s_ref in enumerate(s_refs):
        s = s_ref[0]
        row = b + p * n_steps
        chain_row = chain_ref[row, :]
        m = (chain_row == _ANTIGEN_IDX).astype(jnp.float32).reshape(n_tok, 1)
        tot = jnp.sum(s, axis=0)
        ag = jnp.sum(s * m, axis=0)
        acc_ref[row] = jnp.stack([tot, ag], axis=0)

    @pl.when(b == n_steps - 1)
    def _loss():
        pooled = acc_ref[...]
        mask_all = (chain_ref[...] == _ANTIGEN_IDX).astype(jnp.float32)
        ag_cnt = jnp.sum(mask_all, axis=1)
        ab_cnt = n_tok - ag_cnt

        tot_s = pooled[:, 0, :]
        ag_s = pooled[:, 1, :]
        ab_s = tot_s - ag_s

        ab_emb = ab_s / jnp.maximum(ab_cnt, 1.0)[:, None]
        ag_emb = ag_s / jnp.maximum(ag_cnt, 1.0)[:, None]

        ab_n = ab_emb / jnp.maximum(
            jnp.sqrt(jnp.sum(ab_emb * ab_emb, axis=1, keepdims=True)), 1e-12)
        ag_n = ag_emb / jnp.maximum(
            jnp.sqrt(jnp.sum(ag_emb * ag_emb, axis=1, keepdims=True)), 1e-12)

        sim = jax.lax.dot_general(
            ab_n, ag_n, (((1,), (1,)), ((), ())),
            preferred_element_type=jnp.float32,
            precision=jax.lax.Precision.HIGHEST,
        ) / _TEMPERATURE

        valid = ag_cnt > 0.0
        neg_inf = jnp.asarray(-jnp.inf, dtype=sim.dtype)
        sim_m = jnp.where(valid[None, :], sim, neg_inf)
        mx = jnp.max(sim_m, axis=1, keepdims=True)
        mx_safe = jnp.where(jnp.isfinite(mx), mx, 0.0)
        lse = jnp.log(
            jnp.sum(jnp.exp(sim_m - mx_safe), axis=1, keepdims=True)) + mx

        eye = (jax.lax.broadcasted_iota(jnp.int32, sim.shape, 0)
               == jax.lax.broadcasted_iota(jnp.int32, sim.shape, 1))
        logp = sim - lse
        diag = jnp.sum(jnp.where(eye, logp, 0.0), axis=1)

        n_valid = jnp.sum(valid.astype(jnp.float32))
        loss = -jnp.sum(jnp.where(valid, diag, 0.0)) / n_valid
        out_ref[...] = loss[None, None]


@functools.partial(jax.jit, static_argnames=("interpret",))
def kernel(s_i, chain_type, interpret=False):
    bsz, n_tok, dim = s_i.shape
    n_steps = bsz // _N_PIPE

    def s_spec(p):
        return pl.BlockSpec(
            (1, n_tok, dim), lambda b, p=p: (b + p * n_steps, 0, 0))

    loss = pl.pallas_call(
        _fused_body,
        grid=(n_steps,),
        in_specs=[pl.BlockSpec((bsz, n_tok), lambda b: (0, 0))]
        + [s_spec(p) for p in range(_N_PIPE)],
        out_specs=pl.BlockSpec((1, 1), lambda b: (0, 0)),
        out_shape=jax.ShapeDtypeStruct((1, 1), jnp.float32),
        scratch_shapes=[pltpu.VMEM((bsz, 2, dim), jnp.float32)],
        interpret=interpret,
    )(chain_type, *([s_i] * _N_PIPE))

    return loss[0, 0]
